# baseline (speedup 1.0000x reference)
import functools

import jax
import jax.numpy as jnp
from jax import lax
from jax.experimental import pallas as pl
from jax.experimental.pallas import tpu as pltpu
from jax.experimental.pallas import tpu_sc as plsc

NC = 2
NS = 16
NW = NC * NS
LANES = 128
SUB = 4
CH = SUB * LANES
IG = 6
GR = IG * SUB


def _sc_mesh():
    return plsc.VectorSubcoreMesh(core_axis_name="c", subcore_axis_name="s")


def _make_deg_kernel(npad, nch):
    rows_per_tile = npad // NS

    @functools.partial(
        pl.kernel,
        out_type=jax.ShapeDtypeStruct((NC * npad,), jnp.float32),
        mesh=_sc_mesh(),
        compiler_params=pltpu.CompilerParams(use_tc_tiling_on_sc=False),
        scratch_types=[
            pltpu.VMEM((GR, LANES), jnp.int32),
            pltpu.VMEM((GR * LANES,), jnp.float32),
            pltpu.VMEM((rows_per_tile,), jnp.float32),
            pltpu.VMEM_SHARED((npad,), jnp.float32),
            pltpu.SemaphoreType.DMA,
        ],
    )
    def deg_kernel(dst_hbm, ew_hbm, out_hbm, idx_v, val_v, zb, acc_sh, sem):
        c = lax.axis_index("c")
        s = lax.axis_index("s")
        w = c * NS + s

        def zero_body(i, _):
            zb[pl.ds(i * 16, 16)] = jnp.zeros((16,), jnp.float32)
            return 0

        lax.fori_loop(0, rows_per_tile // 16, zero_body, 0)
        pltpu.sync_copy(zb, acc_sh.at[pl.ds(s * rows_per_tile, rows_per_tile)])
        plsc.subcore_barrier()

        wrows = w * (nch * SUB)

        def group_body(g, _):
            ro = wrows + g * GR
            pltpu.sync_copy(dst_hbm.at[pl.ds(ro, GR)], idx_v)
            pltpu.sync_copy(ew_hbm.at[pl.ds(ro * LANES, GR * LANES)], val_v)
            descs = [
                pltpu.async_copy(val_v.at[pl.ds(k * LANES, LANES)],
                                 acc_sh.at[idx_v.at[k]], sem, add=True)
                for k in range(GR)
            ]
            for d in descs:
                d.wait()
            return 0

        lax.fori_loop(0, nch // IG, group_body, 0)
        plsc.subcore_barrier()
        pltpu.sync_copy(acc_sh.at[pl.ds(s * rows_per_tile, rows_per_tile)],
                        zb)
        pltpu.sync_copy(
            zb,
            out_hbm.at[pl.ds(c * npad + s * rows_per_tile, rows_per_tile)])

    return deg_kernel


def _make_spmm_kernel(npad, nch):
    rows_per_tile = npad // NS

    @functools.partial(
        pl.kernel,
        out_type=jax.ShapeDtypeStruct((NC * npad, 16), jnp.float32),
        mesh=_sc_mesh(),
        compiler_params=pltpu.CompilerParams(use_tc_tiling_on_sc=False),
        scratch_types=[
            pltpu.VMEM((GR, LANES), jnp.int32),
            pltpu.VMEM((GR, LANES), jnp.int32),
            pltpu.VMEM((GR * LANES,), jnp.float32),
            pltpu.VMEM((CH, 16), jnp.float32),
            pltpu.VMEM((CH, 16), jnp.float32),
            pltpu.VMEM_SHARED((npad, 16), jnp.float32),
            pltpu.SemaphoreType.DMA,
            pltpu.SemaphoreType.DMA,
        ],
    )
    def spmm_kernel(src_hbm, dst_hbm, ew_hbm, xd_hbm, out_hbm,
                    idxs_v, idxd_v, ew_v, rows_a, rows_b, acc_sh,
                    gsem, ssem):
        c = lax.axis_index("c")
        s = lax.axis_index("s")
        w = c * NS + s

        def zero_body(i, _):
            rows_a[i] = jnp.zeros((16,), jnp.float32)
            return 0

        lax.fori_loop(0, CH, zero_body, 0)
        base = s * rows_per_tile
        done = 0
        while done < rows_per_tile:
            step = min(CH, rows_per_tile - done)
            pltpu.sync_copy(rows_a.at[pl.ds(0, step)],
                            acc_sh.at[pl.ds(base + done, step)])
            done += step
        plsc.subcore_barrier()

        wrows = w * (nch * SUB)
        bufs = [rows_a, rows_b]

        def fire_gathers(chunk, buf):
            return [
                pltpu.async_copy(
                    xd_hbm.at[idxs_v.at[chunk * SUB + k]],
                    buf.at[pl.ds(k * LANES, LANES)], gsem)
                for k in range(SUB)
            ]

        def fire_scatters(chunk, buf):
            return [
                pltpu.async_copy(
                    buf.at[pl.ds(k * LANES, LANES)],
                    acc_sh.at[idxd_v.at[chunk * SUB + k]], ssem, add=True)
                for k in range(SUB)
            ]

        def scale(chunk, buf):
            def scale_body(g, _):
                ewv = ew_v[pl.ds(chunk * CH + g * 16, 16)]
                for t in range(16):
                    e = g * 16 + t
                    buf[e] = buf[e] * ewv[t]
                return 0

            lax.fori_loop(0, CH // 16, scale_body, 0, unroll=2)

        def group_body(g, _):
            ro = wrows + g * GR
            pltpu.sync_copy(src_hbm.at[pl.ds(ro, GR)], idxs_v)
            pltpu.sync_copy(dst_hbm.at[pl.ds(ro, GR)], idxd_v)
            pltpu.sync_copy(ew_hbm.at[pl.ds(ro * LANES, GR * LANES)], ew_v)
            gd = {0: fire_gathers(0, bufs[0])}
            sd = {}
            for ck in range(IG):
                buf = bufs[ck % 2]
                if ck < IG - 1:
                    if ck >= 1:
                        for d in sd[ck - 1]:
                            d.wait()
                    gd[ck + 1] = fire_gathers(ck + 1, bufs[(ck + 1) % 2])
                for d in gd[ck]:
                    d.wait()
                scale(ck, buf)
                sd[ck] = fire_scatters(ck, buf)
            for d in sd[IG - 2]:
                d.wait()
            for d in sd[IG - 1]:
                d.wait()
            return 0

        lax.fori_loop(0, nch // IG, group_body, 0)
        plsc.subcore_barrier()
        done = 0
        while done < rows_per_tile:
            step = min(CH, rows_per_tile - done)
            pltpu.sync_copy(acc_sh.at[pl.ds(base + done, step)],
                            rows_a.at[pl.ds(0, step)])
            pltpu.sync_copy(
                rows_a.at[pl.ds(0, step)],
                out_hbm.at[pl.ds(c * npad + base + done, step), :])
            done += step

    return spmm_kernel


def _xd_tc_kernel(deg_ref, x_ref, xd_ref):
    deg = deg_ref[0, :] + deg_ref[1, :] + 1.0
    dis = lax.rsqrt(deg)[:, None]
    blk = x_ref.shape[0]
    xd_ref[...] = jnp.concatenate(
        [x_ref[...] * dis, dis, jnp.zeros((blk, 3), jnp.float32)], axis=1)


def _dense_tc_kernel(xt_ref, deg_ref, s_ref, w_ref, o_ref, *, periods,
                     out_dim):
    deg = deg_ref[0] + deg_ref[1] + 1.0
    dis = lax.rsqrt(deg)[None, :]
    sp = s_ref[0] + s_ref[1]
    S = pltpu.einshape("r(jc)->c(rj)", sp, j=8)
    G = dis * (S[:periods, :] + dis * xt_ref[...])
    az = w_ref[:, 0:1]
    cz = w_ref[:, 1:2]
    ah = w_ref[:, 2:3]
    ch = w_ref[:, 3:4]
    blk = G.shape[1]
    acc = jnp.zeros((out_dim, blk), jnp.float32)
    for p in range(periods):
        g = G[p:p + 1, :]
        pk = w_ref[p:p + 1, 5:6]
        acc += pk * (1.0 - jax.nn.sigmoid(g * az + cz)) * jnp.tanh(
            g * ah + ch)
    wl = w_ref[:, 4:5]
    o_ref[...] = (jnp.sum(jnp.maximum(acc, 0.0) * wl, axis=0)
                  + w_ref[0:1, 6])


def kernel(x, edge_index, edge_weight, h, c, attention, Wz, bz, LzW, Lzb,
           Wr, br, LrW, Lrb, Wh, bh, LhW, Lhb, Wl, bl):
    n, periods = x.shape
    out_dim = LzW.shape[1]
    e = edge_weight.shape[0]

    probs = jax.nn.softmax(attention)
    az = (Wz @ LzW[:out_dim]).reshape(out_dim)
    cz = bz @ LzW[:out_dim] + Lzb
    ah = (Wh @ LhW[:out_dim]).reshape(out_dim)
    ch = bh @ LhW[:out_dim] + Lhb
    wmat = jnp.zeros((out_dim, 128), jnp.float32)
    wmat = wmat.at[:, 0].set(az).at[:, 1].set(cz)
    wmat = wmat.at[:, 2].set(ah).at[:, 3].set(ch)
    wmat = wmat.at[:, 4].set(Wl[:, 0])
    wmat = wmat.at[:periods, 5].set(probs)
    wmat = wmat.at[0, 6].set(bl[0])

    nch = -(-e // (NW * CH))
    nch = -(-nch // IG) * IG
    e2 = NW * CH * nch
    pad = e2 - e
    src = jnp.concatenate([edge_index[0], jnp.zeros((pad,), jnp.int32)])
    dst = jnp.concatenate(
        [edge_index[1], jnp.full((pad,), n, jnp.int32)])
    ew = jnp.concatenate([edge_weight, jnp.zeros((pad,), jnp.float32)])
    src2 = src.reshape(-1, LANES)
    dst2 = dst.reshape(-1, LANES)

    npad = -(-(n + 1) // 128) * 128
    xt = x.T

    deg_part = _make_deg_kernel(npad, nch)(dst2, ew).reshape(NC, npad)

    blk = 1024
    grid = (-(-npad // blk),)
    xd = pl.pallas_call(
        _xd_tc_kernel,
        grid=grid,
        in_specs=[
            pl.BlockSpec((NC, blk), lambda i: (0, i)),
            pl.BlockSpec((blk, periods), lambda i: (i, 0)),
        ],
        out_specs=pl.BlockSpec((blk, 16), lambda i: (i, 0)),
        out_shape=jax.ShapeDtypeStruct((n, 16), jnp.float32),
    )(deg_part, x)

    s_part = _make_spmm_kernel(npad, nch)(src2, dst2, ew, xd)
    s_pack = s_part.reshape(NC, npad // 8, 128)

    out = pl.pallas_call(
        functools.partial(_dense_tc_kernel, periods=periods,
                          out_dim=out_dim),
        grid=grid,
        in_specs=[
            pl.BlockSpec((periods, blk), lambda i: (0, i)),
            pl.BlockSpec((NC, blk), lambda i: (0, i)),
            pl.BlockSpec((NC, blk // 8, 128), lambda i: (0, i, 0)),
            pl.BlockSpec((out_dim, 128), lambda i: (0, 0)),
        ],
        out_specs=pl.BlockSpec((blk,), lambda i: (i,)),
        out_shape=jax.ShapeDtypeStruct((npad,), jnp.float32),
    )(xt, deg_part, s_pack, wmat)
    return out[:n].reshape(n, 1)


# speedup vs baseline: 1.4167x; 1.4167x over previous
import functools

import jax
import jax.numpy as jnp
from jax import lax
from jax.experimental import pallas as pl
from jax.experimental.pallas import tpu as pltpu
from jax.experimental.pallas import tpu_sc as plsc

NC = 2
NS = 16
NW = NC * NS
LANES = 128
SUB = 8
CH = SUB * LANES


def _sc_mesh():
    return plsc.VectorSubcoreMesh(core_axis_name="c", subcore_axis_name="s")


def _make_deg_kernel(npad, nch):
    rows_per_tile = npad // NS

    @functools.partial(
        pl.kernel,
        out_type=jax.ShapeDtypeStruct((NC * npad,), jnp.float32),
        mesh=_sc_mesh(),
        compiler_params=pltpu.CompilerParams(use_tc_tiling_on_sc=False),
        scratch_types=[
            pltpu.VMEM((SUB, LANES), jnp.int32),
            pltpu.VMEM((CH,), jnp.float32),
            pltpu.VMEM((rows_per_tile,), jnp.float32),
            pltpu.VMEM_SHARED((npad,), jnp.float32),
            pltpu.SemaphoreType.DMA,
        ],
    )
    def deg_kernel(dst_hbm, ew_hbm, out_hbm, idx_v, val_v, zb, acc_sh, sem):
        c = lax.axis_index("c")
        s = lax.axis_index("s")
        w = c * NS + s

        def zero_body(i, _):
            zb[pl.ds(i * 16, 16)] = jnp.zeros((16,), jnp.float32)
            return 0

        lax.fori_loop(0, rows_per_tile // 16, zero_body, 0)
        pltpu.sync_copy(zb, acc_sh.at[pl.ds(s * rows_per_tile, rows_per_tile)])
        plsc.subcore_barrier()

        wrows = w * (nch * SUB)

        def chunk_body(j, _):
            ro = wrows + j * SUB
            pltpu.sync_copy(dst_hbm.at[pl.ds(ro, SUB)], idx_v)
            pltpu.sync_copy(ew_hbm.at[pl.ds(ro * LANES, CH)], val_v)
            descs = [
                pltpu.async_copy(val_v.at[pl.ds(k * LANES, LANES)],
                                 acc_sh.at[idx_v.at[k]], sem, add=True)
                for k in range(SUB)
            ]
            for d in descs:
                d.wait()
            return 0

        lax.fori_loop(0, nch, chunk_body, 0)
        plsc.subcore_barrier()
        pltpu.sync_copy(acc_sh.at[pl.ds(s * rows_per_tile, rows_per_tile)],
                        zb)
        pltpu.sync_copy(
            zb,
            out_hbm.at[pl.ds(c * npad + s * rows_per_tile, rows_per_tile)])

    return deg_kernel


def _make_spmm_kernel(npad, nch):
    rows_per_tile = npad // NS

    @functools.partial(
        pl.kernel,
        out_type=jax.ShapeDtypeStruct((NC * npad, 16), jnp.float32),
        mesh=_sc_mesh(),
        compiler_params=pltpu.CompilerParams(use_tc_tiling_on_sc=False),
        scratch_types=[
            pltpu.VMEM((SUB, LANES), jnp.int32),
            pltpu.VMEM((SUB, LANES), jnp.int32),
            pltpu.VMEM((CH,), jnp.float32),
            pltpu.VMEM((CH, 16), jnp.float32),
            pltpu.VMEM_SHARED((npad, 16), jnp.float32),
            pltpu.SemaphoreType.DMA,
            pltpu.SemaphoreType.DMA,
        ],
    )
    def spmm_kernel(src_hbm, dst_hbm, ew_hbm, xd_hbm, out_hbm,
                    idxs_v, idxd_v, ew_v, rows_v, acc_sh, gsem, ssem):
        c = lax.axis_index("c")
        s = lax.axis_index("s")
        w = c * NS + s

        def zero_body(i, _):
            rows_v[i] = jnp.zeros((16,), jnp.float32)
            return 0

        lax.fori_loop(0, CH, zero_body, 0)
        base = s * rows_per_tile
        done = 0
        while done < rows_per_tile:
            step = min(CH, rows_per_tile - done)
            pltpu.sync_copy(rows_v.at[pl.ds(0, step)],
                            acc_sh.at[pl.ds(base + done, step)])
            done += step
        plsc.subcore_barrier()

        wrows = w * (nch * SUB)

        def chunk_body(j, _):
            ro = wrows + j * SUB
            pltpu.sync_copy(src_hbm.at[pl.ds(ro, SUB)], idxs_v)
            pltpu.sync_copy(dst_hbm.at[pl.ds(ro, SUB)], idxd_v)
            pltpu.sync_copy(ew_hbm.at[pl.ds(ro * LANES, CH)], ew_v)
            gd = [
                pltpu.async_copy(xd_hbm.at[idxs_v.at[k]],
                                 rows_v.at[pl.ds(k * LANES, LANES)], gsem)
                for k in range(SUB)
            ]
            for d in gd:
                d.wait()

            def scale_body(g, _):
                ewv = ew_v[pl.ds(g * 16, 16)]
                for t in range(16):
                    e = g * 16 + t
                    rows_v[e] = rows_v[e] * ewv[t]
                return 0

            lax.fori_loop(0, CH // 16, scale_body, 0)
            sd = [
                pltpu.async_copy(rows_v.at[pl.ds(k * LANES, LANES)],
                                 acc_sh.at[idxd_v.at[k]], ssem, add=True)
                for k in range(SUB)
            ]
            for d in sd:
                d.wait()
            return 0

        lax.fori_loop(0, nch, chunk_body, 0)
        plsc.subcore_barrier()
        done = 0
        while done < rows_per_tile:
            step = min(CH, rows_per_tile - done)
            pltpu.sync_copy(acc_sh.at[pl.ds(base + done, step)],
                            rows_v.at[pl.ds(0, step)])
            pltpu.sync_copy(
                rows_v.at[pl.ds(0, step)],
                out_hbm.at[pl.ds(c * npad + base + done, step), :])
            done += step

    return spmm_kernel


def _xd_tc_kernel(deg_ref, x_ref, xd_ref):
    deg = deg_ref[0, :] + deg_ref[1, :] + 1.0
    dis = lax.rsqrt(deg)[:, None]
    blk = x_ref.shape[0]
    xd_ref[...] = jnp.concatenate(
        [x_ref[...] * dis, dis, jnp.zeros((blk, 3), jnp.float32)], axis=1)


def _dense_tc_kernel(xt_ref, deg_ref, s_ref, w_ref, o_ref, *, periods,
                     out_dim):
    deg = deg_ref[0] + deg_ref[1] + 1.0
    dis = lax.rsqrt(deg)[None, :]
    sp = s_ref[0] + s_ref[1]
    S = pltpu.einshape("r(jc)->c(rj)", sp, j=8)
    G = dis * (S[:periods, :] + dis * xt_ref[...])
    az = w_ref[:, 0:1]
    cz = w_ref[:, 1:2]
    ah = w_ref[:, 2:3]
    ch = w_ref[:, 3:4]
    blk = G.shape[1]
    acc = jnp.zeros((out_dim, blk), jnp.float32)
    for p in range(periods):
        g = G[p:p + 1, :]
        pk = w_ref[p:p + 1, 5:6]
        acc += pk * (1.0 - jax.nn.sigmoid(g * az + cz)) * jnp.tanh(
            g * ah + ch)
    wl = w_ref[:, 4:5]
    o_ref[...] = (jnp.sum(jnp.maximum(acc, 0.0) * wl, axis=0)
                  + w_ref[0:1, 6])


def kernel(x, edge_index, edge_weight, h, c, attention, Wz, bz, LzW, Lzb,
           Wr, br, LrW, Lrb, Wh, bh, LhW, Lhb, Wl, bl):
    n, periods = x.shape
    out_dim = LzW.shape[1]
    e = edge_weight.shape[0]

    probs = jax.nn.softmax(attention)
    az = (Wz @ LzW[:out_dim]).reshape(out_dim)
    cz = bz @ LzW[:out_dim] + Lzb
    ah = (Wh @ LhW[:out_dim]).reshape(out_dim)
    ch = bh @ LhW[:out_dim] + Lhb
    wmat = jnp.zeros((out_dim, 128), jnp.float32)
    wmat = wmat.at[:, 0].set(az).at[:, 1].set(cz)
    wmat = wmat.at[:, 2].set(ah).at[:, 3].set(ch)
    wmat = wmat.at[:, 4].set(Wl[:, 0])
    wmat = wmat.at[:periods, 5].set(probs)
    wmat = wmat.at[0, 6].set(bl[0])

    nch = -(-e // (NW * CH))
    e2 = NW * CH * nch
    pad = e2 - e
    src = jnp.concatenate([edge_index[0], jnp.zeros((pad,), jnp.int32)])
    dst = jnp.concatenate(
        [edge_index[1], jnp.full((pad,), n, jnp.int32)])
    ew = jnp.concatenate([edge_weight, jnp.zeros((pad,), jnp.float32)])
    src2 = src.reshape(-1, LANES)
    dst2 = dst.reshape(-1, LANES)

    npad = -(-(n + 1) // 128) * 128
    xt = x.T

    deg_part = _make_deg_kernel(npad, nch)(dst2, ew).reshape(NC, npad)

    blk = 1024
    grid = (-(-npad // blk),)
    xd = pl.pallas_call(
        _xd_tc_kernel,
        grid=grid,
        in_specs=[
            pl.BlockSpec((NC, blk), lambda i: (0, i)),
            pl.BlockSpec((blk, periods), lambda i: (i, 0)),
        ],
        out_specs=pl.BlockSpec((blk, 16), lambda i: (i, 0)),
        out_shape=jax.ShapeDtypeStruct((n, 16), jnp.float32),
    )(deg_part, x)

    s_part = _make_spmm_kernel(npad, nch)(src2, dst2, ew, xd)
    s_pack = s_part.reshape(NC, npad // 8, 128)

    out = pl.pallas_call(
        functools.partial(_dense_tc_kernel, periods=periods,
                          out_dim=out_dim),
        grid=grid,
        in_specs=[
            pl.BlockSpec((periods, blk), lambda i: (0, i)),
            pl.BlockSpec((NC, blk), lambda i: (0, i)),
            pl.BlockSpec((NC, blk // 8, 128), lambda i: (0, i, 0)),
            pl.BlockSpec((out_dim, 128), lambda i: (0, 0)),
        ],
        out_specs=pl.BlockSpec((blk,), lambda i: (i,)),
        out_shape=jax.ShapeDtypeStruct((npad,), jnp.float32),
    )(xt, deg_part, s_pack, wmat)
    return out[:n].reshape(n, 1)
